# baseline (speedup 1.0000x reference)
import functools

import jax
import jax.numpy as jnp
from jax import lax
from jax.experimental import pallas as pl
from jax.experimental.pallas import tpu as pltpu
from jax.experimental.pallas import tpu_sc as plsc

N = 10000
E = 320000
D = 128

NC = 2
NS = 16
NW = NC * NS
K = 64
NBUF = 4
EW = 10240
NCHUNK = EW // K
FAST_CORE = 0
CH_FAST = 320
CH_SLOW = 2 * NCHUNK - CH_FAST
QMAX = CH_FAST // 5
KD = 128
E_PAD = NW * EW
R = 10112
RS = R // NS
TRASH = N

_MESH = plsc.VectorSubcoreMesh(core_axis_name="c", subcore_axis_name="s")


def _zero_vmem(buf, rows, cols):
  def body(r, carry):
    for cc in range(cols // 16):
      buf[r, pl.ds(cc * 16, 16)] = jnp.zeros((16,), jnp.float32)
    return carry
  lax.fori_loop(0, rows, body, 0)


def _zero_acc_slice(acc, zbuf, s, zr):
  base = s * RS
  offs = list(range(0, RS - zr + 1, zr))
  if offs[-1] + zr < RS:
    offs.append(RS - zr)
  for off in offs:
    pltpu.sync_copy(zbuf, acc.at[pl.ds(base + off, zr)])


def _seg_body(g2_hbm, src_hbm, dst_hbm, out_hbm,
              acc, r0, r1, r2, r3, sidx, didx, s0, s1, s2, s3):
  c = lax.axis_index("c")
  s = lax.axis_index("s")
  w = c * NS + s
  g_hbm = g2_hbm.at[c]
  rows = (r0, r1, r2, r3)
  sems = (s0, s1, s2, s3)

  _zero_vmem(r0, K, D)
  _zero_acc_slice(acc, r0, s, K)

  plsc.subcore_barrier()

  def run_side(nch, qtr, corebase):
    if nch == 0:
      return
    for part in range(nch // qtr):
      base = corebase + s * nch + part * qtr
      pltpu.sync_copy(src_hbm.at[pl.ds(base, qtr)], sidx.at[pl.ds(0, qtr)])
      pltpu.sync_copy(dst_hbm.at[pl.ds(base, qtr)], didx.at[pl.ds(0, qtr)])
      for b in range(NBUF):
        pltpu.make_async_copy(g_hbm.at[sidx.at[b]], rows[b], sems[b]).start()

      def body(i, carry):
        for b in range(NBUF):
          ch = NBUF * i + b
          pltpu.make_async_copy(g_hbm.at[sidx.at[ch]], rows[b], sems[b]).wait()
          pltpu.sync_copy(rows[b], acc.at[didx.at[ch]], add=True)

          @pl.when(ch + NBUF < qtr)
          def _():
            pltpu.make_async_copy(
                g_hbm.at[sidx.at[ch + NBUF]], rows[b], sems[b]).start()
        return carry

      lax.fori_loop(0, qtr // NBUF, body, 0)

  @pl.when(c == FAST_CORE)
  def _():
    run_side(CH_FAST, CH_FAST // 5, 0 if FAST_CORE == 0 else NS * CH_SLOW)

  @pl.when(c == 1 - FAST_CORE)
  def _():
    run_side(CH_SLOW, CH_SLOW // 5, 0 if FAST_CORE == 1 else NS * CH_FAST)

  plsc.subcore_barrier()
  pltpu.sync_copy(acc.at[pl.ds(s * RS, RS)], out_hbm.at[c, pl.ds(s * RS, RS)])


_seg = pl.kernel(
    _seg_body,
    out_type=jax.ShapeDtypeStruct((NC, R, D), jnp.float32),
    mesh=_MESH,
    scratch_types=[
        pltpu.VMEM_SHARED((R, D), jnp.float32),
        pltpu.VMEM((K, D), jnp.float32),
        pltpu.VMEM((K, D), jnp.float32),
        pltpu.VMEM((K, D), jnp.float32),
        pltpu.VMEM((K, D), jnp.float32),
        pltpu.VMEM((QMAX, K), jnp.int32),
        pltpu.VMEM((QMAX, K), jnp.int32),
        pltpu.SemaphoreType.DMA,
        pltpu.SemaphoreType.DMA,
        pltpu.SemaphoreType.DMA,
        pltpu.SemaphoreType.DMA,
    ],
)


NCH_D = EW // KD


def _deg_body(dst_hbm, out_hbm, dacc, ones, zbuf, didx):
  c = lax.axis_index("c")
  s = lax.axis_index("s")
  w = c * NS + s

  def fill_ones(r, carry):
    for cc in range(D // 16):
      ones[r, pl.ds(cc * 16, 16)] = jnp.full((16,), 1.0, jnp.float32)
    return carry
  lax.fori_loop(0, KD, fill_ones, 0)
  _zero_vmem(zbuf, KD, D)
  _zero_acc_slice(dacc, zbuf, s, KD)

  pltpu.sync_copy(dst_hbm.at[pl.ds(w * NCH_D, NCH_D)], didx)
  plsc.subcore_barrier()

  def body(j, carry):
    pltpu.sync_copy(ones, dacc.at[didx.at[j]], add=True)
    return carry
  lax.fori_loop(0, NCH_D, body, 0)

  plsc.subcore_barrier()
  pltpu.sync_copy(dacc.at[pl.ds(s * RS, RS)], out_hbm.at[c, pl.ds(s * RS, RS)])


_deg = pl.kernel(
    _deg_body,
    out_type=jax.ShapeDtypeStruct((NC, R, D), jnp.float32),
    mesh=_MESH,
    scratch_types=[
        pltpu.VMEM_SHARED((R, D), jnp.float32),
        pltpu.VMEM((KD, D), jnp.float32),
        pltpu.VMEM((KD, D), jnp.float32),
        pltpu.VMEM((NCH_D, KD), jnp.int32),
    ],
)


BLK = 1000


def _proj_tc(h_ref, wt_ref, o_ref):
  o_ref[...] = jnp.dot(h_ref[...], wt_ref[...],
                       preferred_element_type=jnp.float32,
                       precision=lax.Precision.HIGHEST)


def _comb_tc(relu, project, h_ref, p_ref, d_ref, wst_ref, b_ref, wnt_ref,
             o_ref, g_ref):
  neigh = p_ref[0] + p_ref[1]
  denom = jnp.maximum(d_ref[0, :, :1] + d_ref[1, :, :1], 1.0)
  h = jnp.dot(h_ref[...], wst_ref[...],
              preferred_element_type=jnp.float32,
              precision=lax.Precision.HIGHEST)
  h = h + neigh / denom + b_ref[...]
  if relu:
    h = jnp.maximum(h, 0.0)
  o_ref[...] = h
  if project:
    g_ref[...] = jnp.dot(h, wnt_ref[...],
                         preferred_element_type=jnp.float32,
                         precision=lax.Precision.HIGHEST)


def _proj(h, wnt):
  return pl.pallas_call(
      _proj_tc,
      grid=(N // BLK,),
      in_specs=[
          pl.BlockSpec((BLK, D), lambda i: (i, 0)),
          pl.BlockSpec((D, D), lambda i: (0, 0)),
      ],
      out_specs=pl.BlockSpec((BLK, D), lambda i: (i, 0)),
      out_shape=jax.ShapeDtypeStruct((N, D), jnp.float32),
  )(h, wnt)


def _combine(h, P, degp, wst, b, wnt, relu):
  project = wnt is not None
  if not project:
    wnt = wst
  out_shapes = (jax.ShapeDtypeStruct((N, D), jnp.float32),
                jax.ShapeDtypeStruct((N, D), jnp.float32))
  outs = pl.pallas_call(
      functools.partial(_comb_tc, relu, project),
      grid=(N // BLK,),
      in_specs=[
          pl.BlockSpec((BLK, D), lambda i: (i, 0)),
          pl.BlockSpec((NC, BLK, D), lambda i: (0, i, 0)),
          pl.BlockSpec((NC, BLK, D), lambda i: (0, i, 0)),
          pl.BlockSpec((D, D), lambda i: (0, 0)),
          pl.BlockSpec((1, D), lambda i: (0, 0)),
          pl.BlockSpec((D, D), lambda i: (0, 0)),
      ],
      out_specs=(pl.BlockSpec((BLK, D), lambda i: (i, 0)),
                 pl.BlockSpec((BLK, D), lambda i: (i, 0))),
      out_shape=out_shapes,
  )(h, P, degp, wst, b.reshape(1, D), wnt)
  return outs if project else outs[0]


def kernel(feats, edge_index, W_self_0, W_neigh_0, b_0,
           W_self_1, W_neigh_1, b_1, W_self_2, W_neigh_2, b_2):
  src_flat = jnp.concatenate([edge_index[0], jnp.zeros((E_PAD - E,), jnp.int32)])
  dst_flat = jnp.concatenate(
      [edge_index[1], jnp.full((E_PAD - E,), TRASH, jnp.int32)])
  src = src_flat.reshape(-1, K)
  dst = dst_flat.reshape(-1, K)

  degp = _deg(dst_flat.reshape(-1, KD))

  g = _proj(feats, W_neigh_0.T)
  h = feats
  params = [(W_self_0, b_0, W_neigh_1), (W_self_1, b_1, W_neigh_2),
            (W_self_2, b_2, None)]
  for i, (ws, b, wn_next) in enumerate(params):
    P = _seg(jnp.stack([g, g]), src, dst)
    res = _combine(h, P, degp, ws.T, b,
                   None if wn_next is None else wn_next.T, relu=(i < 2))
    if wn_next is None:
      h = res
    else:
      h, g = res
  return h


# speedup vs baseline: 1.1393x; 1.1393x over previous
import functools

import jax
import jax.numpy as jnp
from jax import lax
from jax.experimental import pallas as pl
from jax.experimental.pallas import tpu as pltpu
from jax.experimental.pallas import tpu_sc as plsc

N = 10000
E = 320000
D = 128

NC = 2
NS = 16
NW = NC * NS
K = 64
NBUF = 4
EW = 10240
NCHUNK = EW // K
FAST_CORE = 1
CH_FAST = 240
CH_SLOW = 2 * NCHUNK - CH_FAST
QMAX = CH_FAST // 5
KD = 128
E_PAD = NW * EW
R = 10112
RS = R // NS
TRASH = N

_MESH = plsc.VectorSubcoreMesh(core_axis_name="c", subcore_axis_name="s")


def _zero_vmem(buf, rows, cols):
  def body(r, carry):
    for cc in range(cols // 16):
      buf[r, pl.ds(cc * 16, 16)] = jnp.zeros((16,), jnp.float32)
    return carry
  lax.fori_loop(0, rows, body, 0)


def _zero_acc_slice(acc, zbuf, s, zr):
  base = s * RS
  offs = list(range(0, RS - zr + 1, zr))
  if offs[-1] + zr < RS:
    offs.append(RS - zr)
  for off in offs:
    pltpu.sync_copy(zbuf, acc.at[pl.ds(base + off, zr)])


def _seg_body(g2_hbm, src_hbm, dst_hbm, out_hbm,
              acc, r0, r1, r2, r3, sidx, didx, s0, s1, s2, s3):
  c = lax.axis_index("c")
  s = lax.axis_index("s")
  w = c * NS + s
  g_hbm = g2_hbm.at[c]
  rows = (r0, r1, r2, r3)
  sems = (s0, s1, s2, s3)

  _zero_vmem(r0, K, D)
  _zero_acc_slice(acc, r0, s, K)

  plsc.subcore_barrier()

  def run_side(nch, qtr, corebase):
    if nch == 0:
      return
    for part in range(nch // qtr):
      base = corebase + s * nch + part * qtr
      pltpu.sync_copy(src_hbm.at[pl.ds(base, qtr)], sidx.at[pl.ds(0, qtr)])
      pltpu.sync_copy(dst_hbm.at[pl.ds(base, qtr)], didx.at[pl.ds(0, qtr)])
      for b in range(NBUF):
        pltpu.make_async_copy(g_hbm.at[sidx.at[b]], rows[b], sems[b]).start()

      def body(i, carry):
        for b in range(NBUF):
          ch = NBUF * i + b
          pltpu.make_async_copy(g_hbm.at[sidx.at[ch]], rows[b], sems[b]).wait()
          pltpu.sync_copy(rows[b], acc.at[didx.at[ch]], add=True)

          @pl.when(ch + NBUF < qtr)
          def _():
            pltpu.make_async_copy(
                g_hbm.at[sidx.at[ch + NBUF]], rows[b], sems[b]).start()
        return carry

      lax.fori_loop(0, qtr // NBUF, body, 0)

  @pl.when(c == FAST_CORE)
  def _():
    run_side(CH_FAST, CH_FAST // 5, 0 if FAST_CORE == 0 else NS * CH_SLOW)

  @pl.when(c == 1 - FAST_CORE)
  def _():
    run_side(CH_SLOW, CH_SLOW // 5, 0 if FAST_CORE == 1 else NS * CH_FAST)

  plsc.subcore_barrier()
  pltpu.sync_copy(acc.at[pl.ds(s * RS, RS)], out_hbm.at[c, pl.ds(s * RS, RS)])


_seg = pl.kernel(
    _seg_body,
    out_type=jax.ShapeDtypeStruct((NC, R, D), jnp.float32),
    mesh=_MESH,
    scratch_types=[
        pltpu.VMEM_SHARED((R, D), jnp.float32),
        pltpu.VMEM((K, D), jnp.float32),
        pltpu.VMEM((K, D), jnp.float32),
        pltpu.VMEM((K, D), jnp.float32),
        pltpu.VMEM((K, D), jnp.float32),
        pltpu.VMEM((QMAX, K), jnp.int32),
        pltpu.VMEM((QMAX, K), jnp.int32),
        pltpu.SemaphoreType.DMA,
        pltpu.SemaphoreType.DMA,
        pltpu.SemaphoreType.DMA,
        pltpu.SemaphoreType.DMA,
    ],
)


NCH_D = EW // KD


def _deg_body(dst_hbm, out_hbm, dacc, ones, zbuf, didx):
  c = lax.axis_index("c")
  s = lax.axis_index("s")
  w = c * NS + s

  def fill_ones(r, carry):
    for cc in range(D // 16):
      ones[r, pl.ds(cc * 16, 16)] = jnp.full((16,), 1.0, jnp.float32)
    return carry
  lax.fori_loop(0, KD, fill_ones, 0)
  _zero_vmem(zbuf, KD, D)
  _zero_acc_slice(dacc, zbuf, s, KD)

  pltpu.sync_copy(dst_hbm.at[pl.ds(w * NCH_D, NCH_D)], didx)
  plsc.subcore_barrier()

  def body(j, carry):
    pltpu.sync_copy(ones, dacc.at[didx.at[j]], add=True)
    return carry
  lax.fori_loop(0, NCH_D, body, 0)

  plsc.subcore_barrier()
  pltpu.sync_copy(dacc.at[pl.ds(s * RS, RS)], out_hbm.at[c, pl.ds(s * RS, RS)])


_deg = pl.kernel(
    _deg_body,
    out_type=jax.ShapeDtypeStruct((NC, R, D), jnp.float32),
    mesh=_MESH,
    scratch_types=[
        pltpu.VMEM_SHARED((R, D), jnp.float32),
        pltpu.VMEM((KD, D), jnp.float32),
        pltpu.VMEM((KD, D), jnp.float32),
        pltpu.VMEM((NCH_D, KD), jnp.int32),
    ],
)


BLK = 1000


def _proj_tc(h_ref, wt_ref, o_ref):
  o_ref[...] = jnp.dot(h_ref[...], wt_ref[...],
                       preferred_element_type=jnp.float32,
                       precision=lax.Precision.HIGHEST)


def _comb_tc(relu, project, h_ref, p_ref, d_ref, wst_ref, b_ref, wnt_ref,
             o_ref, g_ref):
  neigh = p_ref[0] + p_ref[1]
  denom = jnp.maximum(d_ref[0, :, :1] + d_ref[1, :, :1], 1.0)
  h = jnp.dot(h_ref[...], wst_ref[...],
              preferred_element_type=jnp.float32,
              precision=lax.Precision.HIGHEST)
  h = h + neigh / denom + b_ref[...]
  if relu:
    h = jnp.maximum(h, 0.0)
  o_ref[...] = h
  if project:
    g_ref[...] = jnp.dot(h, wnt_ref[...],
                         preferred_element_type=jnp.float32,
                         precision=lax.Precision.HIGHEST)


def _proj(h, wnt):
  return pl.pallas_call(
      _proj_tc,
      grid=(N // BLK,),
      in_specs=[
          pl.BlockSpec((BLK, D), lambda i: (i, 0)),
          pl.BlockSpec((D, D), lambda i: (0, 0)),
      ],
      out_specs=pl.BlockSpec((BLK, D), lambda i: (i, 0)),
      out_shape=jax.ShapeDtypeStruct((N, D), jnp.float32),
  )(h, wnt)


def _combine(h, P, degp, wst, b, wnt, relu):
  project = wnt is not None
  if not project:
    wnt = wst
  out_shapes = (jax.ShapeDtypeStruct((N, D), jnp.float32),
                jax.ShapeDtypeStruct((N, D), jnp.float32))
  outs = pl.pallas_call(
      functools.partial(_comb_tc, relu, project),
      grid=(N // BLK,),
      in_specs=[
          pl.BlockSpec((BLK, D), lambda i: (i, 0)),
          pl.BlockSpec((NC, BLK, D), lambda i: (0, i, 0)),
          pl.BlockSpec((NC, BLK, D), lambda i: (0, i, 0)),
          pl.BlockSpec((D, D), lambda i: (0, 0)),
          pl.BlockSpec((1, D), lambda i: (0, 0)),
          pl.BlockSpec((D, D), lambda i: (0, 0)),
      ],
      out_specs=(pl.BlockSpec((BLK, D), lambda i: (i, 0)),
                 pl.BlockSpec((BLK, D), lambda i: (i, 0))),
      out_shape=out_shapes,
  )(h, P, degp, wst, b.reshape(1, D), wnt)
  return outs if project else outs[0]


def kernel(feats, edge_index, W_self_0, W_neigh_0, b_0,
           W_self_1, W_neigh_1, b_1, W_self_2, W_neigh_2, b_2):
  src_flat = jnp.concatenate([edge_index[0], jnp.zeros((E_PAD - E,), jnp.int32)])
  dst_flat = jnp.concatenate(
      [edge_index[1], jnp.full((E_PAD - E,), TRASH, jnp.int32)])
  src = src_flat.reshape(-1, K)
  dst = dst_flat.reshape(-1, K)

  degp = _deg(dst_flat.reshape(-1, KD))

  g = _proj(feats, W_neigh_0.T)
  h = feats
  params = [(W_self_0, b_0, W_neigh_1), (W_self_1, b_1, W_neigh_2),
            (W_self_2, b_2, None)]
  for i, (ws, b, wn_next) in enumerate(params):
    P = _seg(jnp.stack([g, g]), src, dst)
    res = _combine(h, P, degp, ws.T, b,
                   None if wn_next is None else wn_next.T, relu=(i < 2))
    if wn_next is None:
      h = res
    else:
      h, g = res
  return h


# speedup vs baseline: 1.2436x; 1.0915x over previous
import functools

import jax
import jax.numpy as jnp
from jax import lax
from jax.experimental import pallas as pl
from jax.experimental.pallas import tpu as pltpu
from jax.experimental.pallas import tpu_sc as plsc

N = 10000
E = 320000
D = 128

NC = 2
NS = 16
NW = NC * NS
K = 64
NBUF = 4
EW = 10240
NCHUNK = EW // K
FAST_CORE = 0
CH_FAST = 240
CH_SLOW = 2 * NCHUNK - CH_FAST
QMAX = CH_FAST // 5
KD = 128
E_PAD = NW * EW
R = 10112
RS = R // NS
TRASH = N

_MESH = plsc.VectorSubcoreMesh(core_axis_name="c", subcore_axis_name="s")


def _zero_vmem(buf, rows, cols):
  def body(r, carry):
    for cc in range(cols // 16):
      buf[r, pl.ds(cc * 16, 16)] = jnp.zeros((16,), jnp.float32)
    return carry
  lax.fori_loop(0, rows, body, 0)


def _zero_acc_slice(acc, zbuf, s, zr):
  base = s * RS
  offs = list(range(0, RS - zr + 1, zr))
  if offs[-1] + zr < RS:
    offs.append(RS - zr)
  for off in offs:
    pltpu.sync_copy(zbuf, acc.at[pl.ds(base + off, zr)])


def _seg_body(g2_hbm, src_hbm, dst_hbm, out_hbm,
              acc, r0, r1, r2, r3, sidx, didx,
              s0, s1, s2, s3, t0, t1, t2, t3):
  c = lax.axis_index("c")
  s = lax.axis_index("s")
  g_hbm = g2_hbm.at[c]
  rows = (r0, r1, r2, r3)
  sems = (s0, s1, s2, s3)
  ssems = (t0, t1, t2, t3)

  _zero_vmem(r0, K, D)
  _zero_acc_slice(acc, r0, s, K)

  plsc.subcore_barrier()

  def run_side(nch, qtr, corebase):
    if nch == 0:
      return
    for part in range(nch // qtr):
      base = corebase + s * nch + part * qtr
      pltpu.sync_copy(src_hbm.at[pl.ds(base, qtr)], sidx.at[pl.ds(0, qtr)])
      pltpu.sync_copy(dst_hbm.at[pl.ds(base, qtr)], didx.at[pl.ds(0, qtr)])
      pltpu.make_async_copy(g_hbm.at[sidx.at[0]], rows[0], sems[0]).start()
      pltpu.make_async_copy(g_hbm.at[sidx.at[1]], rows[1], sems[1]).start()

      def body(i, carry):
        for b in range(NBUF):
          ch = NBUF * i + b
          pltpu.make_async_copy(g_hbm.at[sidx.at[ch]], rows[b], sems[b]).wait()
          pltpu.async_copy(rows[b], acc.at[didx.at[ch]], ssems[b], add=True)

          @pl.when(ch + 2 < qtr)
          def _():
            nb = (b + 2) % NBUF

            @pl.when(ch >= 2)
            def _():
              pltpu.make_async_copy(
                  rows[nb], acc.at[didx.at[ch - 2]], ssems[nb]).wait()

            pltpu.make_async_copy(
                g_hbm.at[sidx.at[ch + 2]], rows[nb], sems[nb]).start()
        return carry

      lax.fori_loop(0, qtr // NBUF, body, 0)
      for ch in range(qtr - 4, qtr):
        b = ch % NBUF
        pltpu.make_async_copy(rows[b], acc.at[didx.at[ch]], ssems[b]).wait()

  @pl.when(c == FAST_CORE)
  def _():
    run_side(CH_FAST, CH_FAST // 5, 0 if FAST_CORE == 0 else NS * CH_SLOW)

  @pl.when(c == 1 - FAST_CORE)
  def _():
    run_side(CH_SLOW, CH_SLOW // 5, 0 if FAST_CORE == 1 else NS * CH_FAST)

  plsc.subcore_barrier()
  pltpu.sync_copy(acc.at[pl.ds(s * RS, RS)], out_hbm.at[c, pl.ds(s * RS, RS)])


_seg = pl.kernel(
    _seg_body,
    out_type=jax.ShapeDtypeStruct((NC, R, D), jnp.float32),
    mesh=_MESH,
    scratch_types=[
        pltpu.VMEM_SHARED((R, D), jnp.float32),
        pltpu.VMEM((K, D), jnp.float32),
        pltpu.VMEM((K, D), jnp.float32),
        pltpu.VMEM((K, D), jnp.float32),
        pltpu.VMEM((K, D), jnp.float32),
        pltpu.VMEM((QMAX, K), jnp.int32),
        pltpu.VMEM((QMAX, K), jnp.int32),
        pltpu.SemaphoreType.DMA,
        pltpu.SemaphoreType.DMA,
        pltpu.SemaphoreType.DMA,
        pltpu.SemaphoreType.DMA,
        pltpu.SemaphoreType.DMA,
        pltpu.SemaphoreType.DMA,
        pltpu.SemaphoreType.DMA,
        pltpu.SemaphoreType.DMA,
    ],
)


NCH_D = EW // KD


def _deg_body(dst_hbm, out_hbm, dacc, ones, zbuf, didx):
  c = lax.axis_index("c")
  s = lax.axis_index("s")
  w = c * NS + s

  def fill_ones(r, carry):
    for cc in range(D // 16):
      ones[r, pl.ds(cc * 16, 16)] = jnp.full((16,), 1.0, jnp.float32)
    return carry
  lax.fori_loop(0, KD, fill_ones, 0)
  _zero_vmem(zbuf, KD, D)
  _zero_acc_slice(dacc, zbuf, s, KD)

  pltpu.sync_copy(dst_hbm.at[pl.ds(w * NCH_D, NCH_D)], didx)
  plsc.subcore_barrier()

  def body(j, carry):
    pltpu.sync_copy(ones, dacc.at[didx.at[j]], add=True)
    return carry
  lax.fori_loop(0, NCH_D, body, 0)

  plsc.subcore_barrier()
  pltpu.sync_copy(dacc.at[pl.ds(s * RS, RS)], out_hbm.at[c, pl.ds(s * RS, RS)])


_deg = pl.kernel(
    _deg_body,
    out_type=jax.ShapeDtypeStruct((NC, R, D), jnp.float32),
    mesh=_MESH,
    scratch_types=[
        pltpu.VMEM_SHARED((R, D), jnp.float32),
        pltpu.VMEM((KD, D), jnp.float32),
        pltpu.VMEM((KD, D), jnp.float32),
        pltpu.VMEM((NCH_D, KD), jnp.int32),
    ],
)


BLK = 1000


def _proj_tc(h_ref, wt_ref, o_ref):
  o_ref[...] = jnp.dot(h_ref[...], wt_ref[...],
                       preferred_element_type=jnp.float32,
                       precision=lax.Precision.HIGHEST)


def _comb_tc(relu, project, h_ref, p_ref, d_ref, wst_ref, b_ref, wnt_ref,
             o_ref, g_ref):
  neigh = p_ref[0] + p_ref[1]
  denom = jnp.maximum(d_ref[0, :, :1] + d_ref[1, :, :1], 1.0)
  h = jnp.dot(h_ref[...], wst_ref[...],
              preferred_element_type=jnp.float32,
              precision=lax.Precision.HIGHEST)
  h = h + neigh / denom + b_ref[...]
  if relu:
    h = jnp.maximum(h, 0.0)
  o_ref[...] = h
  if project:
    g_ref[...] = jnp.dot(h, wnt_ref[...],
                         preferred_element_type=jnp.float32,
                         precision=lax.Precision.HIGHEST)


def _proj(h, wnt):
  return pl.pallas_call(
      _proj_tc,
      grid=(N // BLK,),
      in_specs=[
          pl.BlockSpec((BLK, D), lambda i: (i, 0)),
          pl.BlockSpec((D, D), lambda i: (0, 0)),
      ],
      out_specs=pl.BlockSpec((BLK, D), lambda i: (i, 0)),
      out_shape=jax.ShapeDtypeStruct((N, D), jnp.float32),
  )(h, wnt)


def _combine(h, P, degp, wst, b, wnt, relu):
  project = wnt is not None
  if not project:
    wnt = wst
  out_shapes = (jax.ShapeDtypeStruct((N, D), jnp.float32),
                jax.ShapeDtypeStruct((N, D), jnp.float32))
  outs = pl.pallas_call(
      functools.partial(_comb_tc, relu, project),
      grid=(N // BLK,),
      in_specs=[
          pl.BlockSpec((BLK, D), lambda i: (i, 0)),
          pl.BlockSpec((NC, BLK, D), lambda i: (0, i, 0)),
          pl.BlockSpec((NC, BLK, D), lambda i: (0, i, 0)),
          pl.BlockSpec((D, D), lambda i: (0, 0)),
          pl.BlockSpec((1, D), lambda i: (0, 0)),
          pl.BlockSpec((D, D), lambda i: (0, 0)),
      ],
      out_specs=(pl.BlockSpec((BLK, D), lambda i: (i, 0)),
                 pl.BlockSpec((BLK, D), lambda i: (i, 0))),
      out_shape=out_shapes,
  )(h, P, degp, wst, b.reshape(1, D), wnt)
  return outs if project else outs[0]


def kernel(feats, edge_index, W_self_0, W_neigh_0, b_0,
           W_self_1, W_neigh_1, b_1, W_self_2, W_neigh_2, b_2):
  src_flat = jnp.concatenate([edge_index[0], jnp.zeros((E_PAD - E,), jnp.int32)])
  dst_flat = jnp.concatenate(
      [edge_index[1], jnp.full((E_PAD - E,), TRASH, jnp.int32)])
  src = src_flat.reshape(-1, K)
  dst = dst_flat.reshape(-1, K)

  degp = _deg(dst_flat.reshape(-1, KD))

  def pack(g):
    return jnp.stack([g, g])

  g = _proj(feats, W_neigh_0.T)
  h = feats
  params = [(W_self_0, b_0, W_neigh_1), (W_self_1, b_1, W_neigh_2),
            (W_self_2, b_2, None)]
  for i, (ws, b, wn_next) in enumerate(params):
    P = _seg(pack(g), src, dst)
    res = _combine(h, P, degp, ws.T, b,
                   None if wn_next is None else wn_next.T, relu=(i < 2))
    if wn_next is None:
      h = res
    else:
      h, g = res
  return h


# speedup vs baseline: 1.2647x; 1.0170x over previous
import functools

import jax
import jax.numpy as jnp
from jax import lax
from jax.experimental import pallas as pl
from jax.experimental.pallas import tpu as pltpu
from jax.experimental.pallas import tpu_sc as plsc

N = 10000
E = 320000
D = 128

NC = 2
NS = 16
NW = NC * NS
K = 64
NBUF = 4
EW = 10240
NCHUNK = EW // K
FAST_CORE = 0
CH_FAST = 256
CH_SLOW = 2 * NCHUNK - CH_FAST
QMAX = 64


def _qtr(nch):
  for q in (64, 48, 40, 32, 24, 16, 8):
    if nch % q == 0:
      return q
  raise ValueError(nch)
KD = 128
E_PAD = NW * EW
R = 10112
RS = R // NS
TRASH = N

_MESH = plsc.VectorSubcoreMesh(core_axis_name="c", subcore_axis_name="s")


def _zero_vmem(buf, rows, cols):
  def body(r, carry):
    for cc in range(cols // 16):
      buf[r, pl.ds(cc * 16, 16)] = jnp.zeros((16,), jnp.float32)
    return carry
  lax.fori_loop(0, rows, body, 0)


def _zero_acc_slice(acc, zbuf, s, zr):
  base = s * RS
  offs = list(range(0, RS - zr + 1, zr))
  if offs[-1] + zr < RS:
    offs.append(RS - zr)
  for off in offs:
    pltpu.sync_copy(zbuf, acc.at[pl.ds(base + off, zr)])


def _seg_body(g2_hbm, src_hbm, dst_hbm, out_hbm,
              acc, r0, r1, r2, r3, sidx, didx,
              s0, s1, s2, s3, t0, t1, t2, t3):
  c = lax.axis_index("c")
  s = lax.axis_index("s")
  g_hbm = g2_hbm.at[c]
  rows = (r0, r1, r2, r3)
  sems = (s0, s1, s2, s3)
  ssems = (t0, t1, t2, t3)

  _zero_vmem(r0, K, D)
  _zero_acc_slice(acc, r0, s, K)

  plsc.subcore_barrier()

  def run_side(nch, qtr, corebase):
    if nch == 0:
      return
    for part in range(nch // qtr):
      base = corebase + s * nch + part * qtr
      pltpu.sync_copy(src_hbm.at[pl.ds(base, qtr)], sidx.at[pl.ds(0, qtr)])
      pltpu.sync_copy(dst_hbm.at[pl.ds(base, qtr)], didx.at[pl.ds(0, qtr)])
      pltpu.make_async_copy(g_hbm.at[sidx.at[0]], rows[0], sems[0]).start()
      pltpu.make_async_copy(g_hbm.at[sidx.at[1]], rows[1], sems[1]).start()

      def body(i, carry):
        for b in range(NBUF):
          ch = NBUF * i + b
          pltpu.make_async_copy(g_hbm.at[sidx.at[ch]], rows[b], sems[b]).wait()
          pltpu.async_copy(rows[b], acc.at[didx.at[ch]], ssems[b], add=True)

          @pl.when(ch + 2 < qtr)
          def _():
            nb = (b + 2) % NBUF

            @pl.when(ch >= 2)
            def _():
              pltpu.make_async_copy(
                  rows[nb], acc.at[didx.at[ch - 2]], ssems[nb]).wait()

            pltpu.make_async_copy(
                g_hbm.at[sidx.at[ch + 2]], rows[nb], sems[nb]).start()
        return carry

      lax.fori_loop(0, qtr // NBUF, body, 0)
      for ch in range(qtr - 4, qtr):
        b = ch % NBUF
        pltpu.make_async_copy(rows[b], acc.at[didx.at[ch]], ssems[b]).wait()

  @pl.when(c == FAST_CORE)
  def _():
    run_side(CH_FAST, _qtr(CH_FAST), 0 if FAST_CORE == 0 else NS * CH_SLOW)

  @pl.when(c == 1 - FAST_CORE)
  def _():
    run_side(CH_SLOW, _qtr(CH_SLOW), 0 if FAST_CORE == 1 else NS * CH_FAST)

  plsc.subcore_barrier()
  pltpu.sync_copy(acc.at[pl.ds(s * RS, RS)], out_hbm.at[c, pl.ds(s * RS, RS)])


_seg = pl.kernel(
    _seg_body,
    out_type=jax.ShapeDtypeStruct((NC, R, D), jnp.float32),
    mesh=_MESH,
    scratch_types=[
        pltpu.VMEM_SHARED((R, D), jnp.float32),
        pltpu.VMEM((K, D), jnp.float32),
        pltpu.VMEM((K, D), jnp.float32),
        pltpu.VMEM((K, D), jnp.float32),
        pltpu.VMEM((K, D), jnp.float32),
        pltpu.VMEM((QMAX, K), jnp.int32),
        pltpu.VMEM((QMAX, K), jnp.int32),
        pltpu.SemaphoreType.DMA,
        pltpu.SemaphoreType.DMA,
        pltpu.SemaphoreType.DMA,
        pltpu.SemaphoreType.DMA,
        pltpu.SemaphoreType.DMA,
        pltpu.SemaphoreType.DMA,
        pltpu.SemaphoreType.DMA,
        pltpu.SemaphoreType.DMA,
    ],
)


NCH_D = EW // KD


def _deg_body(dst_hbm, out_hbm, dacc, ones, zbuf, didx):
  c = lax.axis_index("c")
  s = lax.axis_index("s")
  w = c * NS + s

  def fill_ones(r, carry):
    for cc in range(D // 16):
      ones[r, pl.ds(cc * 16, 16)] = jnp.full((16,), 1.0, jnp.float32)
    return carry
  lax.fori_loop(0, KD, fill_ones, 0)
  _zero_vmem(zbuf, KD, D)
  _zero_acc_slice(dacc, zbuf, s, KD)

  pltpu.sync_copy(dst_hbm.at[pl.ds(w * NCH_D, NCH_D)], didx)
  plsc.subcore_barrier()

  def body(j, carry):
    pltpu.sync_copy(ones, dacc.at[didx.at[j]], add=True)
    return carry
  lax.fori_loop(0, NCH_D, body, 0)

  plsc.subcore_barrier()
  pltpu.sync_copy(dacc.at[pl.ds(s * RS, RS)], out_hbm.at[c, pl.ds(s * RS, RS)])


_deg = pl.kernel(
    _deg_body,
    out_type=jax.ShapeDtypeStruct((NC, R, D), jnp.float32),
    mesh=_MESH,
    scratch_types=[
        pltpu.VMEM_SHARED((R, D), jnp.float32),
        pltpu.VMEM((KD, D), jnp.float32),
        pltpu.VMEM((KD, D), jnp.float32),
        pltpu.VMEM((NCH_D, KD), jnp.int32),
    ],
)


BLK = 1000


def _proj_tc(h_ref, wt_ref, o_ref):
  o_ref[...] = jnp.dot(h_ref[...], wt_ref[...],
                       preferred_element_type=jnp.float32,
                       precision=lax.Precision.HIGHEST)


def _comb_tc(relu, project, h_ref, p_ref, d_ref, wst_ref, b_ref, wnt_ref,
             o_ref, g_ref):
  neigh = p_ref[0] + p_ref[1]
  denom = jnp.maximum(d_ref[0, :, :1] + d_ref[1, :, :1], 1.0)
  h = jnp.dot(h_ref[...], wst_ref[...],
              preferred_element_type=jnp.float32,
              precision=lax.Precision.HIGHEST)
  h = h + neigh / denom + b_ref[...]
  if relu:
    h = jnp.maximum(h, 0.0)
  o_ref[...] = h
  if project:
    g_ref[...] = jnp.dot(h, wnt_ref[...],
                         preferred_element_type=jnp.float32,
                         precision=lax.Precision.HIGHEST)


def _proj(h, wnt):
  return pl.pallas_call(
      _proj_tc,
      grid=(N // BLK,),
      in_specs=[
          pl.BlockSpec((BLK, D), lambda i: (i, 0)),
          pl.BlockSpec((D, D), lambda i: (0, 0)),
      ],
      out_specs=pl.BlockSpec((BLK, D), lambda i: (i, 0)),
      out_shape=jax.ShapeDtypeStruct((N, D), jnp.float32),
  )(h, wnt)


def _combine(h, P, degp, wst, b, wnt, relu):
  project = wnt is not None
  if not project:
    wnt = wst
  out_shapes = (jax.ShapeDtypeStruct((N, D), jnp.float32),
                jax.ShapeDtypeStruct((N, D), jnp.float32))
  outs = pl.pallas_call(
      functools.partial(_comb_tc, relu, project),
      grid=(N // BLK,),
      in_specs=[
          pl.BlockSpec((BLK, D), lambda i: (i, 0)),
          pl.BlockSpec((NC, BLK, D), lambda i: (0, i, 0)),
          pl.BlockSpec((NC, BLK, D), lambda i: (0, i, 0)),
          pl.BlockSpec((D, D), lambda i: (0, 0)),
          pl.BlockSpec((1, D), lambda i: (0, 0)),
          pl.BlockSpec((D, D), lambda i: (0, 0)),
      ],
      out_specs=(pl.BlockSpec((BLK, D), lambda i: (i, 0)),
                 pl.BlockSpec((BLK, D), lambda i: (i, 0))),
      out_shape=out_shapes,
  )(h, P, degp, wst, b.reshape(1, D), wnt)
  return outs if project else outs[0]


def kernel(feats, edge_index, W_self_0, W_neigh_0, b_0,
           W_self_1, W_neigh_1, b_1, W_self_2, W_neigh_2, b_2):
  src_flat = jnp.concatenate([edge_index[0], jnp.zeros((E_PAD - E,), jnp.int32)])
  dst_flat = jnp.concatenate(
      [edge_index[1], jnp.full((E_PAD - E,), TRASH, jnp.int32)])
  src = src_flat.reshape(-1, K)
  dst = dst_flat.reshape(-1, K)

  degp = _deg(dst_flat.reshape(-1, KD))

  def pack(g):
    return jnp.stack([g, g])

  g = _proj(feats, W_neigh_0.T)
  h = feats
  params = [(W_self_0, b_0, W_neigh_1), (W_self_1, b_1, W_neigh_2),
            (W_self_2, b_2, None)]
  for i, (ws, b, wn_next) in enumerate(params):
    P = _seg(pack(g), src, dst)
    res = _combine(h, P, degp, ws.T, b,
                   None if wn_next is None else wn_next.T, relu=(i < 2))
    if wn_next is None:
      h = res
    else:
      h, g = res
  return h
